# baseline (speedup 1.0000x reference)
import functools

import jax
import jax.numpy as jnp
from jax import lax
from jax.experimental import pallas as pl
from jax.experimental.pallas import tpu as pltpu
from jax.experimental.pallas import tpu_sc as plsc

NC = 2
NS = 16
NW = NC * NS
CHUNK = 128


def _sc_mesh():
    return plsc.VectorSubcoreMesh(
        core_axis_name="c", subcore_axis_name="s", num_cores=NC, num_subcores=NS
    )


def _sc_degree(dst_i32, n):
    e = dst_i32.shape[0]
    nch = e // CHUNK
    rows_pt = n // NS
    zeros = jnp.zeros((n,), jnp.float32)
    ones = jnp.ones((CHUNK,), jnp.float32)

    @functools.partial(
        pl.kernel,
        out_type=jax.ShapeDtypeStruct((NC * n,), jnp.float32),
        mesh=_sc_mesh(),
        scratch_types=[
            pltpu.VMEM_SHARED((n,), jnp.float32),
            pltpu.VMEM(((e // (CHUNK * NW)) * CHUNK,), jnp.int32),
            pltpu.VMEM((CHUNK,), jnp.float32),
        ],
    )
    def k(dst_hbm, z_hbm, ones_hbm, out_hbm, acc, didx, vals):
        c = lax.axis_index("c")
        s = lax.axis_index("s")
        w = s * NC + c
        epw = (nch // NW) * CHUNK
        nj = nch // NW
        pltpu.sync_copy(ones_hbm, vals)
        pltpu.sync_copy(
            z_hbm.at[pl.ds(s * rows_pt, rows_pt)],
            acc.at[pl.ds(s * rows_pt, rows_pt)],
        )
        pltpu.sync_copy(dst_hbm.at[pl.ds(w * epw, epw)], didx)
        plsc.subcore_barrier()

        def body(j, carry):
            pltpu.sync_copy(vals, acc.at[didx.at[pl.ds(j * CHUNK, CHUNK)]], add=True)
            return carry

        lax.fori_loop(0, nj, body, 0)
        plsc.subcore_barrier()
        pltpu.sync_copy(
            acc.at[pl.ds(s * rows_pt, rows_pt)],
            out_hbm.at[pl.ds(c * n + s * rows_pt, rows_pt)],
        )

    return k(dst_i32, zeros, ones)


def _sc_segsum(g, src_i32, dst_i32):
    n, d = g.shape
    e = src_i32.shape[0]
    nch = e // CHUNK
    assert nch % (2 * NW) == 0
    nj = nch // NW
    epw = nj * CHUNK
    rows_pt = n // NS
    zeros = jnp.zeros((n, d), jnp.float32)

    @functools.partial(
        pl.kernel,
        out_type=jax.ShapeDtypeStruct((NC, n, d), jnp.float32),
        mesh=_sc_mesh(),
        scratch_types=[
            pltpu.VMEM_SHARED((n, d), jnp.float32),
            pltpu.VMEM((epw,), jnp.int32),
            pltpu.VMEM((epw,), jnp.int32),
            pltpu.VMEM((CHUNK, d), jnp.float32),
        ],
    )
    def k(g_hbm, src_hbm, dst_hbm, z_hbm, out_hbm, acc, sidx, didx, rows):
        c = lax.axis_index("c")
        s = lax.axis_index("s")
        w = s * NC + c
        pltpu.sync_copy(
            z_hbm.at[pl.ds(s * rows_pt, rows_pt), :],
            acc.at[pl.ds(s * rows_pt, rows_pt), :],
        )
        pltpu.sync_copy(src_hbm.at[pl.ds(w * epw, epw)], sidx)
        pltpu.sync_copy(dst_hbm.at[pl.ds(w * epw, epw)], didx)
        plsc.subcore_barrier()

        def body(j, carry):
            pltpu.sync_copy(g_hbm.at[sidx.at[pl.ds(j * CHUNK, CHUNK)]], rows)
            pltpu.sync_copy(rows, acc.at[didx.at[pl.ds(j * CHUNK, CHUNK)]], add=True)
            return carry

        lax.fori_loop(0, nj, body, 0)

        plsc.subcore_barrier()
        pltpu.sync_copy(
            acc.at[pl.ds(s * rows_pt, rows_pt), :],
            out_hbm.at[c, pl.ds(s * rows_pt, rows_pt), :],
        )

    return k(g, src_i32, dst_i32, zeros)


def _sc_gather2(emb, src_i32, dst_i32):
    n, d = emb.shape
    e = src_i32.shape[0]
    nch = e // CHUNK

    @functools.partial(
        pl.kernel,
        out_type=jax.ShapeDtypeStruct((2, e, d), jnp.float32),
        mesh=_sc_mesh(),
        scratch_types=[
            pltpu.VMEM(((nch // NW) * CHUNK,), jnp.int32),
            pltpu.VMEM(((nch // NW) * CHUNK,), jnp.int32),
            pltpu.VMEM((CHUNK, d), jnp.float32),
            pltpu.VMEM((CHUNK, d), jnp.float32),
        ],
    )
    def k(emb_hbm, src_hbm, dst_hbm, out_hbm, sidx, didx, rows_a, rows_b):
        c = lax.axis_index("c")
        s = lax.axis_index("s")
        w = s * NC + c
        nj = nch // NW
        epw = nj * CHUNK
        pltpu.sync_copy(src_hbm.at[pl.ds(w * epw, epw)], sidx)
        pltpu.sync_copy(dst_hbm.at[pl.ds(w * epw, epw)], didx)

        def body(j, carry):
            base = w * epw + j * CHUNK
            pltpu.sync_copy(emb_hbm.at[sidx.at[pl.ds(j * CHUNK, CHUNK)]], rows_a)
            pltpu.sync_copy(rows_a, out_hbm.at[0, pl.ds(base, CHUNK), :])
            pltpu.sync_copy(emb_hbm.at[didx.at[pl.ds(j * CHUNK, CHUNK)]], rows_b)
            pltpu.sync_copy(rows_b, out_hbm.at[1, pl.ds(base, CHUNK), :])
            return carry

        lax.fori_loop(0, nj, body, 0)

    return k(emb, src_i32, dst_i32)


def _tc_pre(degp, x, w):
    n, d = x.shape
    h = w.shape[1]

    def body(degp_ref, x_ref, w_ref, dis_ref, g_ref):
        deg = 1.0 + degp_ref[0] + degp_ref[1]
        dis = lax.rsqrt(deg)
        dis_ref[...] = dis
        g_ref[...] = dis * jnp.dot(
            x_ref[...], w_ref[...], preferred_element_type=jnp.float32
        )

    return pl.pallas_call(
        body,
        out_shape=[
            jax.ShapeDtypeStruct((n, 1), jnp.float32),
            jax.ShapeDtypeStruct((n, h), jnp.float32),
        ],
    )(degp, x, w)


def _tc_mid(accp, g, dis, b, w_next):
    n, d = g.shape
    h_dim = w_next.shape[1]

    def body(accp_ref, g_ref, dis_ref, b_ref, w_ref, gn_ref):
        dis = dis_ref[...]
        hval = jnp.maximum(
            dis * (accp_ref[0] + accp_ref[1] + g_ref[...]) + b_ref[...], 0.0
        )
        gn_ref[...] = dis * jnp.dot(
            hval, w_ref[...], preferred_element_type=jnp.float32
        )

    return pl.pallas_call(
        body,
        out_shape=jax.ShapeDtypeStruct((n, h_dim), jnp.float32),
    )(accp, g, dis, b, w_next)


def _tc_final(accp, g, dis, b):
    n, d = g.shape

    def body(accp_ref, g_ref, dis_ref, b_ref, emb_ref):
        dis = dis_ref[...]
        hval = jnp.maximum(
            dis * (accp_ref[0] + accp_ref[1] + g_ref[...]) + b_ref[...], 0.0
        )
        emb_ref[...] = jnp.where(jnp.isnan(hval), 0.0, hval)

    return pl.pallas_call(
        body,
        out_shape=jax.ShapeDtypeStruct((n, d), jnp.float32),
    )(accp, g, dis, b)


def _tc_scorer(gfeat, m1, mb1, m2, mb2, m3, mb3):
    _, e, d = gfeat.shape
    be = 512
    grid = e // be
    two_h = m1.shape[0]

    def body(g_ref, m1_ref, mb1_ref, m2_ref, mb2_ref, m3_ref, mb3_ref, s_ref):
        ga = g_ref[0]
        gb = g_ref[1]
        s1 = jnp.dot(ga, m1_ref[: two_h // 2, :], preferred_element_type=jnp.float32)
        s1 = s1 + jnp.dot(
            gb, m1_ref[two_h // 2 :, :], preferred_element_type=jnp.float32
        )
        s1 = jnp.maximum(s1 + mb1_ref[...], 0.0)
        s2 = jnp.maximum(
            jnp.dot(s1, m2_ref[...], preferred_element_type=jnp.float32)
            + mb2_ref[...],
            0.0,
        )
        s3 = (
            jnp.dot(s2, m3_ref[...], preferred_element_type=jnp.float32)
            + mb3_ref[...]
        )
        s3 = jnp.clip(s3, -1000000.0, 1000000.0)
        s_ref[...] = jnp.where(jnp.isnan(s3), 0.0, s3)

    return pl.pallas_call(
        body,
        grid=(grid,),
        in_specs=[
            pl.BlockSpec((2, be, d), lambda i: (0, i, 0)),
            pl.BlockSpec(m1.shape, lambda i: (0, 0)),
            pl.BlockSpec(mb1.shape, lambda i: (0,)),
            pl.BlockSpec(m2.shape, lambda i: (0, 0)),
            pl.BlockSpec(mb2.shape, lambda i: (0,)),
            pl.BlockSpec(m3.shape, lambda i: (0, 0)),
            pl.BlockSpec(mb3.shape, lambda i: (0,)),
        ],
        out_specs=pl.BlockSpec((be, 1), lambda i: (i, 0)),
        out_shape=jax.ShapeDtypeStruct((e, 1), jnp.float32),
    )(gfeat, m1, mb1, m2, mb2, m3, mb3)


def kernel(x, edge_index, W1, b1, W2, b2, W3, b3, M1, mb1, M2, mb2, M3, mb3):
    n = x.shape[0]
    e = edge_index.shape[1]
    np_ = ((n + NS * 16 - 1) // (NS * 16)) * (NS * 16)
    epc = 2 * NW * CHUNK
    ep = ((e + epc - 1) // epc) * epc
    src = jnp.pad(edge_index[0].astype(jnp.int32), (0, ep - e), constant_values=n)
    dst = jnp.pad(edge_index[1].astype(jnp.int32), (0, ep - e), constant_values=n)
    xp = jnp.pad(x, ((0, np_ - n), (0, 0)))

    degp = _sc_degree(dst, np_).reshape(NC, np_, 1)
    dis, g1 = _tc_pre(degp, xp, W1)

    a1 = _sc_segsum(g1, src, dst)
    g2 = _tc_mid(a1, g1, dis, b1, W2)

    a2 = _sc_segsum(g2, src, dst)
    g3 = _tc_mid(a2, g2, dis, b2, W3)

    a3 = _sc_segsum(g3, src, dst)
    emb = _tc_final(a3, g3, dis, b3)

    gfeat = _sc_gather2(emb, src, dst)
    s = _tc_scorer(gfeat, M1, mb1, M2, mb2, M3, mb3)

    return (s[:e, 0], emb[:n])


# speedup vs baseline: 1.9751x; 1.9751x over previous
import functools

import jax
import jax.numpy as jnp
from jax import lax
from jax.experimental import pallas as pl
from jax.experimental.pallas import tpu as pltpu
from jax.experimental.pallas import tpu_sc as plsc

NC = 2
NS = 16
NW = NC * NS
CHUNK = 128


def _sc_mesh():
    return plsc.VectorSubcoreMesh(
        core_axis_name="c", subcore_axis_name="s", num_cores=NC, num_subcores=NS
    )


def _sc_degree(dst_i32, n):
    e = dst_i32.shape[0]
    nch = e // CHUNK
    rows_pt = n // NS
    zeros = jnp.zeros((n,), jnp.float32)
    ones = jnp.ones((CHUNK,), jnp.float32)

    @functools.partial(
        pl.kernel,
        out_type=jax.ShapeDtypeStruct((NC * n,), jnp.float32),
        mesh=_sc_mesh(),
        scratch_types=[
            pltpu.VMEM_SHARED((n,), jnp.float32),
            pltpu.VMEM((CHUNK,), jnp.int32),
            pltpu.VMEM((CHUNK,), jnp.float32),
        ],
    )
    def k(dst_hbm, z_hbm, ones_hbm, out_hbm, acc, didx, vals):
        c = lax.axis_index("c")
        s = lax.axis_index("s")
        w = s * NC + c
        pltpu.sync_copy(ones_hbm, vals)
        pltpu.sync_copy(
            z_hbm.at[pl.ds(s * rows_pt, rows_pt)],
            acc.at[pl.ds(s * rows_pt, rows_pt)],
        )
        plsc.subcore_barrier()
        nj = (nch - w + NW - 1) // NW

        def body(j, carry):
            ch = w + j * NW
            pltpu.sync_copy(dst_hbm.at[pl.ds(ch * CHUNK, CHUNK)], didx)
            pltpu.sync_copy(vals, acc.at[didx], add=True)
            return carry

        lax.fori_loop(0, nj, body, 0)
        plsc.subcore_barrier()
        pltpu.sync_copy(
            acc.at[pl.ds(s * rows_pt, rows_pt)],
            out_hbm.at[pl.ds(c * n + s * rows_pt, rows_pt)],
        )

    return k(dst_i32, zeros, ones)


def _sc_segsum(g, src_i32, dst_i32):
    n, d = g.shape
    e = src_i32.shape[0]
    nch = e // CHUNK
    rows_pt = n // NS
    zeros = jnp.zeros((n, d), jnp.float32)

    @functools.partial(
        pl.kernel,
        out_type=jax.ShapeDtypeStruct((NC, n, d), jnp.float32),
        mesh=_sc_mesh(),
        scratch_types=[
            pltpu.VMEM_SHARED((n, d), jnp.float32),
            pltpu.VMEM((CHUNK,), jnp.int32),
            pltpu.VMEM((CHUNK,), jnp.int32),
            pltpu.VMEM((CHUNK, d), jnp.float32),
        ],
    )
    def k(g_hbm, src_hbm, dst_hbm, z_hbm, out_hbm, acc, sidx, didx, rows):
        c = lax.axis_index("c")
        s = lax.axis_index("s")
        w = s * NC + c
        pltpu.sync_copy(
            z_hbm.at[pl.ds(s * rows_pt, rows_pt), :],
            acc.at[pl.ds(s * rows_pt, rows_pt), :],
        )
        plsc.subcore_barrier()
        nj = (nch - w + NW - 1) // NW

        def body(j, carry):
            base = (w + j * NW) * CHUNK
            pltpu.sync_copy(src_hbm.at[pl.ds(base, CHUNK)], sidx)
            pltpu.sync_copy(dst_hbm.at[pl.ds(base, CHUNK)], didx)
            pltpu.sync_copy(g_hbm.at[sidx], rows)
            pltpu.sync_copy(rows, acc.at[didx], add=True)
            return carry

        lax.fori_loop(0, nj, body, 0)

        plsc.subcore_barrier()
        pltpu.sync_copy(
            acc.at[pl.ds(s * rows_pt, rows_pt), :],
            out_hbm.at[c, pl.ds(s * rows_pt, rows_pt), :],
        )

    return k(g, src_i32, dst_i32, zeros)


def _sc_gather2(emb, src_i32, dst_i32):
    n, d = emb.shape
    e = src_i32.shape[0]
    nch = e // CHUNK

    @functools.partial(
        pl.kernel,
        out_type=jax.ShapeDtypeStruct((2, e, d), jnp.float32),
        mesh=_sc_mesh(),
        scratch_types=[
            pltpu.VMEM((CHUNK,), jnp.int32),
            pltpu.VMEM((CHUNK,), jnp.int32),
            pltpu.VMEM((CHUNK, d), jnp.float32),
            pltpu.VMEM((CHUNK, d), jnp.float32),
        ],
    )
    def k(emb_hbm, src_hbm, dst_hbm, out_hbm, sidx, didx, rows_a, rows_b):
        c = lax.axis_index("c")
        s = lax.axis_index("s")
        w = s * NC + c
        nj = (nch - w + NW - 1) // NW

        def body(j, carry):
            base = (w + j * NW) * CHUNK
            pltpu.sync_copy(src_hbm.at[pl.ds(base, CHUNK)], sidx)
            pltpu.sync_copy(dst_hbm.at[pl.ds(base, CHUNK)], didx)
            pltpu.sync_copy(emb_hbm.at[sidx], rows_a)
            pltpu.sync_copy(rows_a, out_hbm.at[0, pl.ds(base, CHUNK), :])
            pltpu.sync_copy(emb_hbm.at[didx], rows_b)
            pltpu.sync_copy(rows_b, out_hbm.at[1, pl.ds(base, CHUNK), :])
            return carry

        lax.fori_loop(0, nj, body, 0)

    return k(emb, src_i32, dst_i32)


def _tc_pre(degp, x, w):
    n, d = x.shape
    h = w.shape[1]

    def body(degp_ref, x_ref, w_ref, dis_ref, g_ref):
        deg = 1.0 + degp_ref[0] + degp_ref[1]
        dis = lax.rsqrt(deg)
        dis_ref[...] = dis
        g_ref[...] = dis * jnp.dot(
            x_ref[...], w_ref[...], preferred_element_type=jnp.float32
        )

    return pl.pallas_call(
        body,
        out_shape=[
            jax.ShapeDtypeStruct((n, 1), jnp.float32),
            jax.ShapeDtypeStruct((n, h), jnp.float32),
        ],
    )(degp, x, w)


def _tc_mid(accp, g, dis, b, w_next):
    n, d = g.shape
    h_dim = w_next.shape[1]

    def body(accp_ref, g_ref, dis_ref, b_ref, w_ref, gn_ref):
        dis = dis_ref[...]
        hval = jnp.maximum(
            dis * (accp_ref[0] + accp_ref[1] + g_ref[...]) + b_ref[...], 0.0
        )
        gn_ref[...] = dis * jnp.dot(
            hval, w_ref[...], preferred_element_type=jnp.float32
        )

    return pl.pallas_call(
        body,
        out_shape=jax.ShapeDtypeStruct((n, h_dim), jnp.float32),
    )(accp, g, dis, b, w_next)


def _tc_final(accp, g, dis, b):
    n, d = g.shape

    def body(accp_ref, g_ref, dis_ref, b_ref, emb_ref):
        dis = dis_ref[...]
        hval = jnp.maximum(
            dis * (accp_ref[0] + accp_ref[1] + g_ref[...]) + b_ref[...], 0.0
        )
        emb_ref[...] = jnp.where(jnp.isnan(hval), 0.0, hval)

    return pl.pallas_call(
        body,
        out_shape=jax.ShapeDtypeStruct((n, d), jnp.float32),
    )(accp, g, dis, b)


def _tc_scorer(gfeat, m1, mb1, m2, mb2, m3, mb3):
    _, e, d = gfeat.shape
    be = 512
    grid = e // be
    two_h = m1.shape[0]

    def body(g_ref, m1_ref, mb1_ref, m2_ref, mb2_ref, m3_ref, mb3_ref, s_ref):
        ga = g_ref[0]
        gb = g_ref[1]
        s1 = jnp.dot(ga, m1_ref[: two_h // 2, :], preferred_element_type=jnp.float32)
        s1 = s1 + jnp.dot(
            gb, m1_ref[two_h // 2 :, :], preferred_element_type=jnp.float32
        )
        s1 = jnp.maximum(s1 + mb1_ref[...], 0.0)
        s2 = jnp.maximum(
            jnp.dot(s1, m2_ref[...], preferred_element_type=jnp.float32)
            + mb2_ref[...],
            0.0,
        )
        s3 = (
            jnp.dot(s2, m3_ref[...], preferred_element_type=jnp.float32)
            + mb3_ref[...]
        )
        s3 = jnp.clip(s3, -1000000.0, 1000000.0)
        s_ref[...] = jnp.where(jnp.isnan(s3), 0.0, s3)

    return pl.pallas_call(
        body,
        grid=(grid,),
        in_specs=[
            pl.BlockSpec((2, be, d), lambda i: (0, i, 0)),
            pl.BlockSpec(m1.shape, lambda i: (0, 0)),
            pl.BlockSpec(mb1.shape, lambda i: (0,)),
            pl.BlockSpec(m2.shape, lambda i: (0, 0)),
            pl.BlockSpec(mb2.shape, lambda i: (0,)),
            pl.BlockSpec(m3.shape, lambda i: (0, 0)),
            pl.BlockSpec(mb3.shape, lambda i: (0,)),
        ],
        out_specs=pl.BlockSpec((be, 1), lambda i: (i, 0)),
        out_shape=jax.ShapeDtypeStruct((e, 1), jnp.float32),
    )(gfeat, m1, mb1, m2, mb2, m3, mb3)


def kernel(x, edge_index, W1, b1, W2, b2, W3, b3, M1, mb1, M2, mb2, M3, mb3):
    n = x.shape[0]
    e = edge_index.shape[1]
    np_ = ((n + NS * 16 - 1) // (NS * 16)) * (NS * 16)
    src = edge_index[0].astype(jnp.int32)
    dst = edge_index[1].astype(jnp.int32)
    xp = jnp.pad(x, ((0, np_ - n), (0, 0)))

    degp = _sc_degree(dst, np_).reshape(NC, np_, 1)
    dis, g1 = _tc_pre(degp, xp, W1)

    a1 = _sc_segsum(g1, src, dst)
    g2 = _tc_mid(a1, g1, dis, b1, W2)

    a2 = _sc_segsum(g2, src, dst)
    g3 = _tc_mid(a2, g2, dis, b2, W3)

    a3 = _sc_segsum(g3, src, dst)
    emb = _tc_final(a3, g3, dis, b3)

    gfeat = _sc_gather2(emb, src, dst)
    s = _tc_scorer(gfeat, M1, mb1, M2, mb2, M3, mb3)

    return (s[:e, 0], emb[:n])


# speedup vs baseline: 1.9809x; 1.0029x over previous
import functools

import jax
import jax.numpy as jnp
from jax import lax
from jax.experimental import pallas as pl
from jax.experimental.pallas import tpu as pltpu
from jax.experimental.pallas import tpu_sc as plsc

NC = 2
NS = 16
NW = NC * NS
CHUNK = 128


def _sc_mesh():
    return plsc.VectorSubcoreMesh(
        core_axis_name="c", subcore_axis_name="s", num_cores=NC, num_subcores=NS
    )


def _sc_degree(dst_i32, n):
    e = dst_i32.shape[0]
    nch = e // CHUNK
    rows_pt = n // NS
    zeros = jnp.zeros((n,), jnp.float32)
    ones = jnp.ones((CHUNK,), jnp.float32)

    @functools.partial(
        pl.kernel,
        out_type=jax.ShapeDtypeStruct((NC * n,), jnp.float32),
        mesh=_sc_mesh(),
        scratch_types=[
            pltpu.VMEM_SHARED((n,), jnp.float32),
            pltpu.VMEM((CHUNK,), jnp.int32),
            pltpu.VMEM((CHUNK,), jnp.float32),
        ],
    )
    def k(dst_hbm, z_hbm, ones_hbm, out_hbm, acc, didx, vals):
        c = lax.axis_index("c")
        s = lax.axis_index("s")
        w = s * NC + c
        pltpu.sync_copy(ones_hbm, vals)
        pltpu.sync_copy(
            z_hbm.at[pl.ds(s * rows_pt, rows_pt)],
            acc.at[pl.ds(s * rows_pt, rows_pt)],
        )
        plsc.subcore_barrier()
        nj = (nch - w + NW - 1) // NW

        def body(j, carry):
            ch = w + j * NW
            pltpu.sync_copy(dst_hbm.at[pl.ds(ch * CHUNK, CHUNK)], didx)
            pltpu.sync_copy(vals, acc.at[didx], add=True)
            return carry

        lax.fori_loop(0, nj, body, 0)
        plsc.subcore_barrier()
        pltpu.sync_copy(
            acc.at[pl.ds(s * rows_pt, rows_pt)],
            out_hbm.at[pl.ds(c * n + s * rows_pt, rows_pt)],
        )

    return k(dst_i32, zeros, ones)


def _sc_segsum(g, src_i32, dst_i32):
    n, d = g.shape
    e = src_i32.shape[0]
    nch = e // CHUNK
    rows_pt = n // NS
    zeros = jnp.zeros((n, d), jnp.float32)

    @functools.partial(
        pl.kernel,
        out_type=jax.ShapeDtypeStruct((NC, n, d), jnp.float32),
        mesh=_sc_mesh(),
        scratch_types=[
            pltpu.VMEM_SHARED((n, d), jnp.float32),
            pltpu.VMEM((CHUNK,), jnp.int32),
            pltpu.VMEM((CHUNK,), jnp.int32),
            pltpu.VMEM((CHUNK, d), jnp.float32),
        ],
    )
    def k(g_hbm, src_hbm, dst_hbm, z_hbm, out_hbm, acc, sidx, didx, rows):
        c = lax.axis_index("c")
        s = lax.axis_index("s")
        w = s * NC + c
        pltpu.sync_copy(
            z_hbm.at[pl.ds(s * rows_pt, rows_pt), :],
            acc.at[pl.ds(s * rows_pt, rows_pt), :],
        )
        plsc.subcore_barrier()
        nj = (nch - w + NW - 1) // NW

        def body(j, carry):
            base = (w + j * NW) * CHUNK
            pltpu.sync_copy(src_hbm.at[pl.ds(base, CHUNK)], sidx)
            pltpu.sync_copy(dst_hbm.at[pl.ds(base, CHUNK)], didx)
            pltpu.sync_copy(g_hbm.at[sidx], rows)
            pltpu.sync_copy(rows, acc.at[didx], add=True)
            return carry

        lax.fori_loop(0, nj, body, 0)

        plsc.subcore_barrier()
        pltpu.sync_copy(
            acc.at[pl.ds(s * rows_pt, rows_pt), :],
            out_hbm.at[c, pl.ds(s * rows_pt, rows_pt), :],
        )

    return k(g, src_i32, dst_i32, zeros)


def _sc_gather2(emb, src_i32, dst_i32):
    n, d = emb.shape
    e = src_i32.shape[0]
    nch = e // CHUNK

    @functools.partial(
        pl.kernel,
        out_type=jax.ShapeDtypeStruct((2, e, d), emb.dtype),
        mesh=_sc_mesh(),
        scratch_types=[
            pltpu.VMEM((CHUNK,), jnp.int32),
            pltpu.VMEM((CHUNK,), jnp.int32),
            pltpu.VMEM((CHUNK, d), emb.dtype),
            pltpu.VMEM((CHUNK, d), emb.dtype),
        ],
    )
    def k(emb_hbm, src_hbm, dst_hbm, out_hbm, sidx, didx, rows_a, rows_b):
        c = lax.axis_index("c")
        s = lax.axis_index("s")
        w = s * NC + c
        nj = (nch - w + NW - 1) // NW

        def body(j, carry):
            base = (w + j * NW) * CHUNK
            pltpu.sync_copy(src_hbm.at[pl.ds(base, CHUNK)], sidx)
            pltpu.sync_copy(dst_hbm.at[pl.ds(base, CHUNK)], didx)
            pltpu.sync_copy(emb_hbm.at[sidx], rows_a)
            pltpu.sync_copy(rows_a, out_hbm.at[0, pl.ds(base, CHUNK), :])
            pltpu.sync_copy(emb_hbm.at[didx], rows_b)
            pltpu.sync_copy(rows_b, out_hbm.at[1, pl.ds(base, CHUNK), :])
            return carry

        lax.fori_loop(0, nj, body, 0)

    return k(emb, src_i32, dst_i32)


def _tc_pre(degp, x, w):
    n, d = x.shape
    h = w.shape[1]

    def body(degp_ref, x_ref, w_ref, dis_ref, g_ref):
        deg = 1.0 + degp_ref[0] + degp_ref[1]
        dis = lax.rsqrt(deg)
        dis_ref[...] = dis
        g_ref[...] = dis * jnp.dot(
            x_ref[...], w_ref[...], preferred_element_type=jnp.float32
        )

    return pl.pallas_call(
        body,
        out_shape=[
            jax.ShapeDtypeStruct((n, 1), jnp.float32),
            jax.ShapeDtypeStruct((n, h), jnp.float32),
        ],
    )(degp, x, w)


def _tc_mid(accp, g, dis, b, w_next):
    n, d = g.shape
    h_dim = w_next.shape[1]

    def body(accp_ref, g_ref, dis_ref, b_ref, w_ref, gn_ref):
        dis = dis_ref[...]
        hval = jnp.maximum(
            dis * (accp_ref[0] + accp_ref[1] + g_ref[...]) + b_ref[...], 0.0
        )
        gn_ref[...] = dis * jnp.dot(
            hval, w_ref[...], preferred_element_type=jnp.float32
        )

    return pl.pallas_call(
        body,
        out_shape=jax.ShapeDtypeStruct((n, h_dim), jnp.float32),
    )(accp, g, dis, b, w_next)


def _tc_final(accp, g, dis, b):
    n, d = g.shape

    def body(accp_ref, g_ref, dis_ref, b_ref, emb_ref):
        dis = dis_ref[...]
        hval = jnp.maximum(
            dis * (accp_ref[0] + accp_ref[1] + g_ref[...]) + b_ref[...], 0.0
        )
        hval = jnp.where(jnp.isnan(hval), 0.0, hval)
        emb_ref[...] = hval

    return pl.pallas_call(
        body,
        out_shape=jax.ShapeDtypeStruct((n, d), jnp.float32),
    )(accp, g, dis, b)


def _tc_scorer(gfeat, m1, mb1, m2, mb2, m3, mb3):
    _, e, dp = gfeat.shape
    be = 512
    grid = e // be

    def body(g_ref, m1_ref, mb1_ref, m2_ref, mb2_ref, m3_ref, mb3_ref, s_ref):
        ga = g_ref[0].astype(jnp.bfloat16)
        gb = g_ref[1].astype(jnp.bfloat16)
        s1 = jnp.dot(ga, m1_ref[0], preferred_element_type=jnp.float32)
        s1 = s1 + jnp.dot(gb, m1_ref[1], preferred_element_type=jnp.float32)
        s1 = jnp.maximum(s1 + mb1_ref[...], 0.0)
        s2 = jnp.maximum(
            jnp.dot(s1, m2_ref[...], preferred_element_type=jnp.float32)
            + mb2_ref[...],
            0.0,
        )
        s3 = (
            jnp.dot(s2, m3_ref[...], preferred_element_type=jnp.float32)
            + mb3_ref[...]
        )
        s3 = jnp.clip(s3, -1000000.0, 1000000.0)
        s_ref[...] = jnp.where(jnp.isnan(s3), 0.0, s3)

    return pl.pallas_call(
        body,
        grid=(grid,),
        in_specs=[
            pl.BlockSpec((2, be, dp), lambda i: (0, i, 0)),
            pl.BlockSpec(m1.shape, lambda i: (0, 0, 0)),
            pl.BlockSpec(mb1.shape, lambda i: (0,)),
            pl.BlockSpec(m2.shape, lambda i: (0, 0)),
            pl.BlockSpec(mb2.shape, lambda i: (0,)),
            pl.BlockSpec(m3.shape, lambda i: (0, 0)),
            pl.BlockSpec(mb3.shape, lambda i: (0,)),
        ],
        out_specs=pl.BlockSpec((be, 1), lambda i: (i, 0)),
        out_shape=jax.ShapeDtypeStruct((e, 1), jnp.float32),
    )(gfeat, m1, mb1, m2, mb2, m3, mb3)


def kernel(x, edge_index, W1, b1, W2, b2, W3, b3, M1, mb1, M2, mb2, M3, mb3):
    n = x.shape[0]
    e = edge_index.shape[1]
    np_ = ((n + NS * 16 - 1) // (NS * 16)) * (NS * 16)
    src = edge_index[0].astype(jnp.int32)
    dst = edge_index[1].astype(jnp.int32)
    xp = jnp.pad(x, ((0, np_ - n), (0, 0)))

    degp = _sc_degree(dst, np_).reshape(NC, np_, 1)
    dis, g1 = _tc_pre(degp, xp, W1)

    a1 = _sc_segsum(g1, src, dst)
    g2 = _tc_mid(a1, g1, dis, b1, W2)

    a2 = _sc_segsum(g2, src, dst)
    g3 = _tc_mid(a2, g2, dis, b2, W3)

    a3 = _sc_segsum(g3, src, dst)
    emb = _tc_final(a3, g3, dis, b3)

    gfeat = _sc_gather2(emb, src, dst)
    h2 = M1.shape[0] // 2
    m1s = jnp.stack([M1[:h2], M1[h2:]]).astype(jnp.bfloat16)
    s = _tc_scorer(gfeat, m1s, mb1, M2, mb2, M3, mb3)

    return (s[:e, 0], emb[:n])


# speedup vs baseline: 2.2294x; 1.1255x over previous
import functools

import jax
import jax.numpy as jnp
from jax import lax
from jax.experimental import pallas as pl
from jax.experimental.pallas import tpu as pltpu
from jax.experimental.pallas import tpu_sc as plsc

NC = 2
NS = 16
NW = NC * NS
CHUNK = 128


def _sc_mesh():
    return plsc.VectorSubcoreMesh(
        core_axis_name="c", subcore_axis_name="s", num_cores=NC, num_subcores=NS
    )


def _sc_degree(dst_i32, n):
    e = dst_i32.shape[0]
    nch = e // CHUNK
    rows_pt = n // NS
    zeros = jnp.zeros((n,), jnp.float32)
    ones = jnp.ones((CHUNK,), jnp.float32)

    @functools.partial(
        pl.kernel,
        out_type=jax.ShapeDtypeStruct((NC * n,), jnp.float32),
        mesh=_sc_mesh(),
        scratch_types=[
            pltpu.VMEM_SHARED((n,), jnp.float32),
            pltpu.VMEM((CHUNK,), jnp.int32),
            pltpu.VMEM((CHUNK,), jnp.float32),
        ],
    )
    def k(dst_hbm, z_hbm, ones_hbm, out_hbm, acc, didx, vals):
        c = lax.axis_index("c")
        s = lax.axis_index("s")
        w = s * NC + c
        pltpu.sync_copy(ones_hbm, vals)
        pltpu.sync_copy(
            z_hbm.at[pl.ds(s * rows_pt, rows_pt)],
            acc.at[pl.ds(s * rows_pt, rows_pt)],
        )
        plsc.subcore_barrier()
        nj = (nch - w + NW - 1) // NW

        def body(j, carry):
            ch = w + j * NW
            pltpu.sync_copy(dst_hbm.at[pl.ds(ch * CHUNK, CHUNK)], didx)
            pltpu.sync_copy(vals, acc.at[didx], add=True)
            return carry

        lax.fori_loop(0, nj, body, 0)
        plsc.subcore_barrier()
        pltpu.sync_copy(
            acc.at[pl.ds(s * rows_pt, rows_pt)],
            out_hbm.at[pl.ds(c * n + s * rows_pt, rows_pt)],
        )

    return k(dst_i32, zeros, ones)


def _sc_segsum(g, src_i32, dst_i32):
    n, d = g.shape
    e = src_i32.shape[0]
    nch = e // CHUNK
    rows_pt = n // NS
    zeros = jnp.zeros((n, d), jnp.float32)

    @functools.partial(
        pl.kernel,
        out_type=jax.ShapeDtypeStruct((NC, n, d), jnp.float32),
        mesh=_sc_mesh(),
        scratch_types=[
            pltpu.VMEM_SHARED((n, d), jnp.float32),
            pltpu.VMEM((CHUNK,), jnp.int32),
            pltpu.VMEM((CHUNK,), jnp.int32),
            pltpu.VMEM((CHUNK, d), jnp.float32),
        ],
    )
    def k(g_hbm, src_hbm, dst_hbm, z_hbm, out_hbm, acc, sidx, didx, rows):
        c = lax.axis_index("c")
        s = lax.axis_index("s")
        w = s * NC + c
        pltpu.sync_copy(
            z_hbm.at[pl.ds(s * rows_pt, rows_pt), :],
            acc.at[pl.ds(s * rows_pt, rows_pt), :],
        )
        plsc.subcore_barrier()
        nj = (nch - w + NW - 1) // NW

        def body(j, carry):
            base = (w + j * NW) * CHUNK
            pltpu.sync_copy(src_hbm.at[pl.ds(base, CHUNK)], sidx)
            pltpu.sync_copy(dst_hbm.at[pl.ds(base, CHUNK)], didx)
            pltpu.sync_copy(g_hbm.at[sidx], rows)
            pltpu.sync_copy(rows, acc.at[didx], add=True)
            return carry

        lax.fori_loop(0, nj, body, 0)

        plsc.subcore_barrier()
        pltpu.sync_copy(
            acc.at[pl.ds(s * rows_pt, rows_pt), :],
            out_hbm.at[c, pl.ds(s * rows_pt, rows_pt), :],
        )

    return k(g, src_i32, dst_i32, zeros)


def _sc_gather2(emb, src_i32, dst_i32):
    n, d = emb.shape
    e = src_i32.shape[0]
    nch = e // CHUNK

    @functools.partial(
        pl.kernel,
        out_type=jax.ShapeDtypeStruct((2, e, d), emb.dtype),
        mesh=_sc_mesh(),
        scratch_types=[
            pltpu.VMEM((CHUNK,), jnp.int32),
            pltpu.VMEM((CHUNK,), jnp.int32),
            pltpu.VMEM((CHUNK, d), emb.dtype),
            pltpu.VMEM((CHUNK, d), emb.dtype),
        ],
    )
    def k(emb_hbm, src_hbm, dst_hbm, out_hbm, sidx, didx, rows_a, rows_b):
        c = lax.axis_index("c")
        s = lax.axis_index("s")
        w = s * NC + c
        nj = (nch - w + NW - 1) // NW

        def body(j, carry):
            base = (w + j * NW) * CHUNK
            pltpu.sync_copy(src_hbm.at[pl.ds(base, CHUNK)], sidx)
            pltpu.sync_copy(dst_hbm.at[pl.ds(base, CHUNK)], didx)
            pltpu.sync_copy(emb_hbm.at[sidx], rows_a)
            pltpu.sync_copy(rows_a, out_hbm.at[0, pl.ds(base, CHUNK), :])
            pltpu.sync_copy(emb_hbm.at[didx], rows_b)
            pltpu.sync_copy(rows_b, out_hbm.at[1, pl.ds(base, CHUNK), :])
            return carry

        lax.fori_loop(0, nj, body, 0)

    return k(emb, src_i32, dst_i32)


def _tc_pre(degp, x, w):
    n, d = x.shape
    h = w.shape[1]

    def body(degp_ref, x_ref, w_ref, dis_ref, g_ref):
        deg = 1.0 + degp_ref[0] + degp_ref[1]
        dis = lax.rsqrt(deg)
        dis_ref[...] = dis
        g_ref[...] = dis * jnp.dot(
            x_ref[...], w_ref[...], preferred_element_type=jnp.float32
        )

    return pl.pallas_call(
        body,
        out_shape=[
            jax.ShapeDtypeStruct((n, 1), jnp.float32),
            jax.ShapeDtypeStruct((n, h), jnp.float32),
        ],
    )(degp, x, w)


def _tc_mid(accp, g, dis, b, w_next):
    n, d = g.shape
    h_dim = w_next.shape[1]

    def body(accp_ref, g_ref, dis_ref, b_ref, w_ref, gn_ref):
        dis = dis_ref[...]
        hval = jnp.maximum(
            dis * (accp_ref[0] + accp_ref[1] + g_ref[...]) + b_ref[...], 0.0
        )
        gn_ref[...] = dis * jnp.dot(
            hval, w_ref[...], preferred_element_type=jnp.float32
        )

    return pl.pallas_call(
        body,
        out_shape=jax.ShapeDtypeStruct((n, h_dim), jnp.float32),
    )(accp, g, dis, b, w_next)


def _tc_final(accp, g, dis, b):
    n, d = g.shape

    def body(accp_ref, g_ref, dis_ref, b_ref, emb_ref):
        dis = dis_ref[...]
        hval = jnp.maximum(
            dis * (accp_ref[0] + accp_ref[1] + g_ref[...]) + b_ref[...], 0.0
        )
        hval = jnp.where(jnp.isnan(hval), 0.0, hval)
        emb_ref[...] = hval

    return pl.pallas_call(
        body,
        out_shape=jax.ShapeDtypeStruct((n, d), jnp.float32),
    )(accp, g, dis, b)


def _tc_scorer(gfeat, m1, mb1, m2, mb2, m3, mb3):
    _, e, dp = gfeat.shape
    be = 512
    grid = e // be

    def body(g_ref, m1_ref, mb1_ref, m2_ref, mb2_ref, m3_ref, mb3_ref, s_ref):
        ga = g_ref[0].astype(jnp.bfloat16)
        gb = g_ref[1].astype(jnp.bfloat16)
        s1 = jnp.dot(ga, m1_ref[0], preferred_element_type=jnp.float32)
        s1 = s1 + jnp.dot(gb, m1_ref[1], preferred_element_type=jnp.float32)
        s1 = jnp.maximum(s1 + mb1_ref[...], 0.0)
        s2 = jnp.maximum(
            jnp.dot(s1, m2_ref[...], preferred_element_type=jnp.float32)
            + mb2_ref[...],
            0.0,
        )
        s3 = (
            jnp.dot(s2, m3_ref[...], preferred_element_type=jnp.float32)
            + mb3_ref[...]
        )
        s3 = jnp.clip(s3, -1000000.0, 1000000.0)
        s_ref[...] = jnp.where(jnp.isnan(s3), 0.0, s3)

    return pl.pallas_call(
        body,
        grid=(grid,),
        in_specs=[
            pl.BlockSpec((2, be, dp), lambda i: (0, i, 0)),
            pl.BlockSpec(m1.shape, lambda i: (0, 0, 0)),
            pl.BlockSpec(mb1.shape, lambda i: (0,)),
            pl.BlockSpec(m2.shape, lambda i: (0, 0)),
            pl.BlockSpec(mb2.shape, lambda i: (0,)),
            pl.BlockSpec(m3.shape, lambda i: (0, 0)),
            pl.BlockSpec(mb3.shape, lambda i: (0,)),
        ],
        out_specs=pl.BlockSpec((be, 1), lambda i: (i, 0)),
        out_shape=jax.ShapeDtypeStruct((e, 1), jnp.float32),
    )(gfeat, m1, mb1, m2, mb2, m3, mb3)


def kernel(x, edge_index, W1, b1, W2, b2, W3, b3, M1, mb1, M2, mb2, M3, mb3):
    n = x.shape[0]
    e = edge_index.shape[1]
    np_ = ((n + NS * 16 - 1) // (NS * 16)) * (NS * 16)
    src = edge_index[0].astype(jnp.int32)
    dst = edge_index[1].astype(jnp.int32)
    xp = jnp.pad(x, ((0, np_ - n), (0, 0)))

    degp = _sc_degree(dst, np_).reshape(NC, np_, 1)
    dis, g1 = _tc_pre(degp, xp, W1)

    a1 = _sc_segsum(g1, src, dst)
    g2 = _tc_mid(a1, g1, dis, b1, W2)

    a2 = _sc_segsum(g2, src, dst)
    g3 = _tc_mid(a2, g2, dis, b2, W3)

    a3 = _sc_segsum(g3, src, dst)
    emb = _tc_final(a3, g3, dis, b3)

    h2 = M1.shape[0] // 2
    m1s = jnp.stack([M1[:h2], M1[h2:]]).astype(jnp.bfloat16)
    nsc = 5
    ec = e // nsc
    parts = []
    for k in range(nsc):
        gf = _sc_gather2(emb, src[k * ec : (k + 1) * ec], dst[k * ec : (k + 1) * ec])
        parts.append(_tc_scorer(gf, m1s, mb1, M2, mb2, M3, mb3))
    s = jnp.concatenate(parts, axis=0)

    return (s[:e, 0], emb[:n])


# speedup vs baseline: 2.4888x; 1.1163x over previous
import functools

import jax
import jax.numpy as jnp
from jax import lax
from jax.experimental import pallas as pl
from jax.experimental.pallas import tpu as pltpu
from jax.experimental.pallas import tpu_sc as plsc

NC = 2
NS = 16
NW = NC * NS
CHUNK = 128


def _sc_mesh():
    return plsc.VectorSubcoreMesh(
        core_axis_name="c", subcore_axis_name="s", num_cores=NC, num_subcores=NS
    )


def _sc_degree(dst_i32, n):
    e = dst_i32.shape[0]
    nch = e // CHUNK
    rows_pt = n // NS
    zeros = jnp.zeros((n,), jnp.float32)
    ones = jnp.ones((CHUNK,), jnp.float32)

    @functools.partial(
        pl.kernel,
        out_type=jax.ShapeDtypeStruct((NC * n,), jnp.float32),
        mesh=_sc_mesh(),
        scratch_types=[
            pltpu.VMEM_SHARED((n,), jnp.float32),
            pltpu.VMEM((CHUNK,), jnp.int32),
            pltpu.VMEM((CHUNK,), jnp.float32),
        ],
    )
    def k(dst_hbm, z_hbm, ones_hbm, out_hbm, acc, didx, vals):
        c = lax.axis_index("c")
        s = lax.axis_index("s")
        w = s * NC + c
        pltpu.sync_copy(ones_hbm, vals)
        pltpu.sync_copy(
            z_hbm.at[pl.ds(s * rows_pt, rows_pt)],
            acc.at[pl.ds(s * rows_pt, rows_pt)],
        )
        plsc.subcore_barrier()
        nj = (nch - w + NW - 1) // NW

        def body(j, carry):
            ch = w + j * NW
            pltpu.sync_copy(dst_hbm.at[pl.ds(ch * CHUNK, CHUNK)], didx)
            pltpu.sync_copy(vals, acc.at[didx], add=True)
            return carry

        lax.fori_loop(0, nj, body, 0)
        plsc.subcore_barrier()
        pltpu.sync_copy(
            acc.at[pl.ds(s * rows_pt, rows_pt)],
            out_hbm.at[pl.ds(c * n + s * rows_pt, rows_pt)],
        )

    return k(dst_i32, zeros, ones)


def _sc_segsum(g, src_i32, dst_i32):
    n, d = g.shape
    e = src_i32.shape[0]
    KB = 2
    nch = e // (CHUNK * KB)
    rows_pt = n // NS
    zeros = jnp.zeros((n, d), jnp.float32)

    @functools.partial(
        pl.kernel,
        out_type=jax.ShapeDtypeStruct((NC, n, d), jnp.float32),
        mesh=_sc_mesh(),
        scratch_types=[
            pltpu.VMEM_SHARED((n, d), jnp.float32),
            pltpu.VMEM((KB * CHUNK,), jnp.int32),
            pltpu.VMEM((KB * CHUNK,), jnp.int32),
            pltpu.VMEM((KB * CHUNK, d), jnp.float32),
        ],
    )
    def k(g_hbm, src_hbm, dst_hbm, z_hbm, out_hbm, acc, sidx, didx, rows):
        c = lax.axis_index("c")
        s = lax.axis_index("s")
        w = s * NC + c
        pltpu.sync_copy(
            z_hbm.at[pl.ds(s * rows_pt, rows_pt), :],
            acc.at[pl.ds(s * rows_pt, rows_pt), :],
        )
        plsc.subcore_barrier()
        nj = (nch - w + NW - 1) // NW

        def body(j, carry):
            base = (w + j * NW) * CHUNK * KB
            pltpu.sync_copy(src_hbm.at[pl.ds(base, CHUNK * KB)], sidx)
            pltpu.sync_copy(dst_hbm.at[pl.ds(base, CHUNK * KB)], didx)
            pltpu.sync_copy(g_hbm.at[sidx], rows)
            pltpu.sync_copy(rows, acc.at[didx], add=True)
            return carry

        lax.fori_loop(0, nj, body, 0)

        plsc.subcore_barrier()
        pltpu.sync_copy(
            acc.at[pl.ds(s * rows_pt, rows_pt), :],
            out_hbm.at[c, pl.ds(s * rows_pt, rows_pt), :],
        )

    return k(g, src_i32, dst_i32, zeros)


def _sc_gather2(emb, src_i32, dst_i32):
    n, d = emb.shape
    e = src_i32.shape[0]
    nch = e // CHUNK

    @functools.partial(
        pl.kernel,
        out_type=jax.ShapeDtypeStruct((2, e, d), emb.dtype),
        mesh=_sc_mesh(),
        scratch_types=[
            pltpu.VMEM((CHUNK,), jnp.int32),
            pltpu.VMEM((CHUNK,), jnp.int32),
            pltpu.VMEM((CHUNK, d), emb.dtype),
            pltpu.VMEM((CHUNK, d), emb.dtype),
        ],
    )
    def k(emb_hbm, src_hbm, dst_hbm, out_hbm, sidx, didx, rows_a, rows_b):
        c = lax.axis_index("c")
        s = lax.axis_index("s")
        w = s * NC + c
        nj = (nch - w + NW - 1) // NW

        def body(j, carry):
            base = (w + j * NW) * CHUNK
            pltpu.sync_copy(src_hbm.at[pl.ds(base, CHUNK)], sidx)
            pltpu.sync_copy(dst_hbm.at[pl.ds(base, CHUNK)], didx)
            pltpu.sync_copy(emb_hbm.at[sidx], rows_a)
            pltpu.sync_copy(rows_a, out_hbm.at[0, pl.ds(base, CHUNK), :])
            pltpu.sync_copy(emb_hbm.at[didx], rows_b)
            pltpu.sync_copy(rows_b, out_hbm.at[1, pl.ds(base, CHUNK), :])
            return carry

        lax.fori_loop(0, nj, body, 0)

    return k(emb, src_i32, dst_i32)


def _tc_pre(degp, x, w):
    n, d = x.shape
    h = w.shape[1]

    def body(degp_ref, x_ref, w_ref, dis_ref, g_ref):
        deg = 1.0 + degp_ref[0] + degp_ref[1]
        dis = lax.rsqrt(deg)
        dis_ref[...] = dis
        g_ref[...] = dis * jnp.dot(
            x_ref[...], w_ref[...], preferred_element_type=jnp.float32
        )

    return pl.pallas_call(
        body,
        out_shape=[
            jax.ShapeDtypeStruct((n, 1), jnp.float32),
            jax.ShapeDtypeStruct((n, h), jnp.float32),
        ],
    )(degp, x, w)


def _tc_mid(accp, g, dis, b, w_next):
    n, d = g.shape
    h_dim = w_next.shape[1]

    def body(accp_ref, g_ref, dis_ref, b_ref, w_ref, gn_ref):
        dis = dis_ref[...]
        hval = jnp.maximum(
            dis * (accp_ref[0] + accp_ref[1] + g_ref[...]) + b_ref[...], 0.0
        )
        gn_ref[...] = dis * jnp.dot(
            hval, w_ref[...], preferred_element_type=jnp.float32
        )

    return pl.pallas_call(
        body,
        out_shape=jax.ShapeDtypeStruct((n, h_dim), jnp.float32),
    )(accp, g, dis, b, w_next)


def _tc_final(accp, g, dis, b):
    n, d = g.shape

    def body(accp_ref, g_ref, dis_ref, b_ref, emb_ref):
        dis = dis_ref[...]
        hval = jnp.maximum(
            dis * (accp_ref[0] + accp_ref[1] + g_ref[...]) + b_ref[...], 0.0
        )
        hval = jnp.where(jnp.isnan(hval), 0.0, hval)
        emb_ref[...] = hval

    return pl.pallas_call(
        body,
        out_shape=jax.ShapeDtypeStruct((n, d), jnp.float32),
    )(accp, g, dis, b)


def _tc_scorer(gfeat, m1, mb1, m2, mb2, m3, mb3):
    _, e, dp = gfeat.shape
    be = 512
    grid = e // be

    def body(g_ref, m1_ref, mb1_ref, m2_ref, mb2_ref, m3_ref, mb3_ref, s_ref):
        ga = g_ref[0].astype(jnp.bfloat16)
        gb = g_ref[1].astype(jnp.bfloat16)
        s1 = jnp.dot(ga, m1_ref[0], preferred_element_type=jnp.float32)
        s1 = s1 + jnp.dot(gb, m1_ref[1], preferred_element_type=jnp.float32)
        s1 = jnp.maximum(s1 + mb1_ref[...], 0.0)
        s2 = jnp.maximum(
            jnp.dot(s1, m2_ref[...], preferred_element_type=jnp.float32)
            + mb2_ref[...],
            0.0,
        )
        s3 = (
            jnp.dot(s2, m3_ref[...], preferred_element_type=jnp.float32)
            + mb3_ref[...]
        )
        s3 = jnp.clip(s3, -1000000.0, 1000000.0)
        s_ref[...] = jnp.where(jnp.isnan(s3), 0.0, s3)

    return pl.pallas_call(
        body,
        grid=(grid,),
        in_specs=[
            pl.BlockSpec((2, be, dp), lambda i: (0, i, 0)),
            pl.BlockSpec(m1.shape, lambda i: (0, 0, 0)),
            pl.BlockSpec(mb1.shape, lambda i: (0,)),
            pl.BlockSpec(m2.shape, lambda i: (0, 0)),
            pl.BlockSpec(mb2.shape, lambda i: (0,)),
            pl.BlockSpec(m3.shape, lambda i: (0, 0)),
            pl.BlockSpec(mb3.shape, lambda i: (0,)),
        ],
        out_specs=pl.BlockSpec((be, 1), lambda i: (i, 0)),
        out_shape=jax.ShapeDtypeStruct((e, 1), jnp.float32),
    )(gfeat, m1, mb1, m2, mb2, m3, mb3)


def kernel(x, edge_index, W1, b1, W2, b2, W3, b3, M1, mb1, M2, mb2, M3, mb3):
    n = x.shape[0]
    e = edge_index.shape[1]
    np_ = ((n + NS * 16 - 1) // (NS * 16)) * (NS * 16)
    src = edge_index[0].astype(jnp.int32)
    dst = edge_index[1].astype(jnp.int32)
    xp = jnp.pad(x, ((0, np_ - n), (0, 0)))

    degp = _sc_degree(dst, np_).reshape(NC, np_, 1)
    dis, g1 = _tc_pre(degp, xp, W1)

    a1 = _sc_segsum(g1, src, dst)
    g2 = _tc_mid(a1, g1, dis, b1, W2)

    a2 = _sc_segsum(g2, src, dst)
    g3 = _tc_mid(a2, g2, dis, b2, W3)

    a3 = _sc_segsum(g3, src, dst)
    emb = _tc_final(a3, g3, dis, b3)

    h2 = M1.shape[0] // 2
    m1s = jnp.stack([M1[:h2], M1[h2:]]).astype(jnp.bfloat16)
    nsc = 5
    ec = e // nsc
    parts = []
    for k in range(nsc):
        gf = _sc_gather2(emb, src[k * ec : (k + 1) * ec], dst[k * ec : (k + 1) * ec])
        parts.append(_tc_scorer(gf, m1s, mb1, M2, mb2, M3, mb3))
    s = jnp.concatenate(parts, axis=0)

    return (s[:e, 0], emb[:n])


# speedup vs baseline: 2.5362x; 1.0191x over previous
import functools

import jax
import jax.numpy as jnp
from jax import lax
from jax.experimental import pallas as pl
from jax.experimental.pallas import tpu as pltpu
from jax.experimental.pallas import tpu_sc as plsc

NC = 2
NS = 16
NW = NC * NS
CHUNK = 128


def _sc_mesh():
    return plsc.VectorSubcoreMesh(
        core_axis_name="c", subcore_axis_name="s", num_cores=NC, num_subcores=NS
    )


def _sc_degree(dst_i32, n):
    e = dst_i32.shape[0]
    KB = 2
    nch = e // (CHUNK * KB)
    rows_pt = n // NS
    zeros = jnp.zeros((n,), jnp.float32)
    ones = jnp.ones((KB * CHUNK,), jnp.float32)

    @functools.partial(
        pl.kernel,
        out_type=jax.ShapeDtypeStruct((NC * n,), jnp.float32),
        mesh=_sc_mesh(),
        scratch_types=[
            pltpu.VMEM_SHARED((n,), jnp.float32),
            pltpu.VMEM((KB * CHUNK,), jnp.int32),
            pltpu.VMEM((KB * CHUNK,), jnp.float32),
        ],
    )
    def k(dst_hbm, z_hbm, ones_hbm, out_hbm, acc, didx, vals):
        c = lax.axis_index("c")
        s = lax.axis_index("s")
        w = s * NC + c
        pltpu.sync_copy(ones_hbm, vals)
        pltpu.sync_copy(
            z_hbm.at[pl.ds(s * rows_pt, rows_pt)],
            acc.at[pl.ds(s * rows_pt, rows_pt)],
        )
        plsc.subcore_barrier()
        nj = (nch - w + NW - 1) // NW

        def body(j, carry):
            base = (w + j * NW) * CHUNK * KB
            pltpu.sync_copy(dst_hbm.at[pl.ds(base, CHUNK * KB)], didx)
            pltpu.sync_copy(vals, acc.at[didx], add=True)
            return carry

        lax.fori_loop(0, nj, body, 0)
        plsc.subcore_barrier()
        pltpu.sync_copy(
            acc.at[pl.ds(s * rows_pt, rows_pt)],
            out_hbm.at[pl.ds(c * n + s * rows_pt, rows_pt)],
        )

    return k(dst_i32, zeros, ones)


def _sc_segsum(g, src_i32, dst_i32):
    n, d = g.shape
    e = src_i32.shape[0]
    KB = 2
    nch = e // (CHUNK * KB)
    rows_pt = n // NS
    zeros = jnp.zeros((n, d), jnp.float32)

    @functools.partial(
        pl.kernel,
        out_type=jax.ShapeDtypeStruct((NC, n, d), jnp.float32),
        mesh=_sc_mesh(),
        scratch_types=[
            pltpu.VMEM_SHARED((n, d), jnp.float32),
            pltpu.VMEM((KB * CHUNK,), jnp.int32),
            pltpu.VMEM((KB * CHUNK,), jnp.int32),
            pltpu.VMEM((KB * CHUNK, d), jnp.float32),
        ],
    )
    def k(g_hbm, src_hbm, dst_hbm, z_hbm, out_hbm, acc, sidx, didx, rows):
        c = lax.axis_index("c")
        s = lax.axis_index("s")
        w = s * NC + c
        pltpu.sync_copy(
            z_hbm.at[pl.ds(s * rows_pt, rows_pt), :],
            acc.at[pl.ds(s * rows_pt, rows_pt), :],
        )
        plsc.subcore_barrier()
        nj = (nch - w + NW - 1) // NW

        def body(j, carry):
            base = (w + j * NW) * CHUNK * KB
            pltpu.sync_copy(src_hbm.at[pl.ds(base, CHUNK * KB)], sidx)
            pltpu.sync_copy(dst_hbm.at[pl.ds(base, CHUNK * KB)], didx)
            pltpu.sync_copy(g_hbm.at[sidx], rows)
            pltpu.sync_copy(rows, acc.at[didx], add=True)
            return carry

        lax.fori_loop(0, nj, body, 0)

        plsc.subcore_barrier()
        pltpu.sync_copy(
            acc.at[pl.ds(s * rows_pt, rows_pt), :],
            out_hbm.at[c, pl.ds(s * rows_pt, rows_pt), :],
        )

    return k(g, src_i32, dst_i32, zeros)


def _sc_gather2(emb, src_i32, dst_i32):
    n, d = emb.shape
    e = src_i32.shape[0]
    KB = 2
    nch = e // (CHUNK * KB)

    @functools.partial(
        pl.kernel,
        out_type=jax.ShapeDtypeStruct((2, e, d), emb.dtype),
        mesh=_sc_mesh(),
        scratch_types=[
            pltpu.VMEM((KB * CHUNK,), jnp.int32),
            pltpu.VMEM((KB * CHUNK,), jnp.int32),
            pltpu.VMEM((KB * CHUNK, d), emb.dtype),
        ],
    )
    def k(emb_hbm, src_hbm, dst_hbm, out_hbm, sidx, didx, rows):
        c = lax.axis_index("c")
        s = lax.axis_index("s")
        w = s * NC + c
        nj = (nch - w + NW - 1) // NW

        def body(j, carry):
            base = (w + j * NW) * CHUNK * KB
            pltpu.sync_copy(src_hbm.at[pl.ds(base, CHUNK * KB)], sidx)
            pltpu.sync_copy(dst_hbm.at[pl.ds(base, CHUNK * KB)], didx)
            pltpu.sync_copy(emb_hbm.at[sidx], rows)
            pltpu.sync_copy(rows, out_hbm.at[0, pl.ds(base, CHUNK * KB), :])
            pltpu.sync_copy(emb_hbm.at[didx], rows)
            pltpu.sync_copy(rows, out_hbm.at[1, pl.ds(base, CHUNK * KB), :])
            return carry

        lax.fori_loop(0, nj, body, 0)

    return k(emb, src_i32, dst_i32)


def _tc_pre(degp, x, w):
    n, d = x.shape
    h = w.shape[1]

    def body(degp_ref, x_ref, w_ref, dis_ref, g_ref):
        deg = 1.0 + degp_ref[0] + degp_ref[1]
        dis = lax.rsqrt(deg)
        dis_ref[...] = dis
        g_ref[...] = dis * jnp.dot(
            x_ref[...], w_ref[...], preferred_element_type=jnp.float32
        )

    return pl.pallas_call(
        body,
        out_shape=[
            jax.ShapeDtypeStruct((n, 1), jnp.float32),
            jax.ShapeDtypeStruct((n, h), jnp.float32),
        ],
    )(degp, x, w)


def _tc_mid(accp, g, dis, b, w_next):
    n, d = g.shape
    h_dim = w_next.shape[1]

    def body(accp_ref, g_ref, dis_ref, b_ref, w_ref, gn_ref):
        dis = dis_ref[...]
        hval = jnp.maximum(
            dis * (accp_ref[0] + accp_ref[1] + g_ref[...]) + b_ref[...], 0.0
        )
        gn_ref[...] = dis * jnp.dot(
            hval, w_ref[...], preferred_element_type=jnp.float32
        )

    return pl.pallas_call(
        body,
        out_shape=jax.ShapeDtypeStruct((n, h_dim), jnp.float32),
    )(accp, g, dis, b, w_next)


def _tc_final(accp, g, dis, b):
    n, d = g.shape

    def body(accp_ref, g_ref, dis_ref, b_ref, emb_ref):
        dis = dis_ref[...]
        hval = jnp.maximum(
            dis * (accp_ref[0] + accp_ref[1] + g_ref[...]) + b_ref[...], 0.0
        )
        hval = jnp.where(jnp.isnan(hval), 0.0, hval)
        emb_ref[...] = hval

    return pl.pallas_call(
        body,
        out_shape=jax.ShapeDtypeStruct((n, d), jnp.float32),
    )(accp, g, dis, b)


def _tc_scorer(gfeat, m1, mb1, m2, mb2, m3, mb3):
    _, e, dp = gfeat.shape
    be = 512
    grid = e // be

    def body(g_ref, m1_ref, mb1_ref, m2_ref, mb2_ref, m3_ref, mb3_ref, s_ref):
        ga = g_ref[0].astype(jnp.bfloat16)
        gb = g_ref[1].astype(jnp.bfloat16)
        s1 = jnp.dot(ga, m1_ref[0], preferred_element_type=jnp.float32)
        s1 = s1 + jnp.dot(gb, m1_ref[1], preferred_element_type=jnp.float32)
        s1 = jnp.maximum(s1 + mb1_ref[...], 0.0)
        s2 = jnp.maximum(
            jnp.dot(s1, m2_ref[...], preferred_element_type=jnp.float32)
            + mb2_ref[...],
            0.0,
        )
        s3 = (
            jnp.dot(s2, m3_ref[...], preferred_element_type=jnp.float32)
            + mb3_ref[...]
        )
        s3 = jnp.clip(s3, -1000000.0, 1000000.0)
        s_ref[...] = jnp.where(jnp.isnan(s3), 0.0, s3)

    return pl.pallas_call(
        body,
        grid=(grid,),
        in_specs=[
            pl.BlockSpec((2, be, dp), lambda i: (0, i, 0)),
            pl.BlockSpec(m1.shape, lambda i: (0, 0, 0)),
            pl.BlockSpec(mb1.shape, lambda i: (0,)),
            pl.BlockSpec(m2.shape, lambda i: (0, 0)),
            pl.BlockSpec(mb2.shape, lambda i: (0,)),
            pl.BlockSpec(m3.shape, lambda i: (0, 0)),
            pl.BlockSpec(mb3.shape, lambda i: (0,)),
        ],
        out_specs=pl.BlockSpec((be, 1), lambda i: (i, 0)),
        out_shape=jax.ShapeDtypeStruct((e, 1), jnp.float32),
    )(gfeat, m1, mb1, m2, mb2, m3, mb3)


def kernel(x, edge_index, W1, b1, W2, b2, W3, b3, M1, mb1, M2, mb2, M3, mb3):
    n = x.shape[0]
    e = edge_index.shape[1]
    np_ = ((n + NS * 16 - 1) // (NS * 16)) * (NS * 16)
    src = edge_index[0].astype(jnp.int32)
    dst = edge_index[1].astype(jnp.int32)
    xp = jnp.pad(x, ((0, np_ - n), (0, 0)))

    degp = _sc_degree(dst, np_).reshape(NC, np_, 1)
    dis, g1 = _tc_pre(degp, xp, W1)

    a1 = _sc_segsum(g1, src, dst)
    g2 = _tc_mid(a1, g1, dis, b1, W2)

    a2 = _sc_segsum(g2, src, dst)
    g3 = _tc_mid(a2, g2, dis, b2, W3)

    a3 = _sc_segsum(g3, src, dst)
    emb = _tc_final(a3, g3, dis, b3)

    h2 = M1.shape[0] // 2
    m1s = jnp.stack([M1[:h2], M1[h2:]]).astype(jnp.bfloat16)
    nsc = 5
    ec = e // nsc
    parts = []
    for k in range(nsc):
        gf = _sc_gather2(emb, src[k * ec : (k + 1) * ec], dst[k * ec : (k + 1) * ec])
        parts.append(_tc_scorer(gf, m1s, mb1, M2, mb2, M3, mb3))
    s = jnp.concatenate(parts, axis=0)

    return (s[:e, 0], emb[:n])


# speedup vs baseline: 2.9718x; 1.1718x over previous
import functools

import jax
import jax.numpy as jnp
from jax import lax
from jax.experimental import pallas as pl
from jax.experimental.pallas import tpu as pltpu
from jax.experimental.pallas import tpu_sc as plsc

NC = 2
NS = 16
NW = NC * NS
CHUNK = 128


def _sc_mesh():
    return plsc.VectorSubcoreMesh(
        core_axis_name="c", subcore_axis_name="s", num_cores=NC, num_subcores=NS
    )


def _sc_degree(dst_i32, n):
    e = dst_i32.shape[0]
    KB = 2
    nch = e // (CHUNK * KB)
    rows_pt = n // NS
    zeros = jnp.zeros((n,), jnp.float32)
    ones = jnp.ones((KB * CHUNK,), jnp.float32)

    @functools.partial(
        pl.kernel,
        out_type=jax.ShapeDtypeStruct((NC * n,), jnp.float32),
        mesh=_sc_mesh(),
        scratch_types=[
            pltpu.VMEM_SHARED((n,), jnp.float32),
            pltpu.VMEM((KB * CHUNK,), jnp.int32),
            pltpu.VMEM((KB * CHUNK,), jnp.float32),
        ],
    )
    def k(dst_hbm, z_hbm, ones_hbm, out_hbm, acc, didx, vals):
        c = lax.axis_index("c")
        s = lax.axis_index("s")
        w = s * NC + c
        pltpu.sync_copy(ones_hbm, vals)
        pltpu.sync_copy(
            z_hbm.at[pl.ds(s * rows_pt, rows_pt)],
            acc.at[pl.ds(s * rows_pt, rows_pt)],
        )
        plsc.subcore_barrier()
        nj = (nch - w + NW - 1) // NW

        def body(j, carry):
            base = (w + j * NW) * CHUNK * KB
            pltpu.sync_copy(dst_hbm.at[pl.ds(base, CHUNK * KB)], didx)
            pltpu.sync_copy(vals, acc.at[didx], add=True)
            return carry

        lax.fori_loop(0, nj, body, 0)
        plsc.subcore_barrier()
        pltpu.sync_copy(
            acc.at[pl.ds(s * rows_pt, rows_pt)],
            out_hbm.at[pl.ds(c * n + s * rows_pt, rows_pt)],
        )

    return k(dst_i32, zeros, ones)


def _sc_segsum(g, src_i32, dst_i32):
    n, d = g.shape
    e = src_i32.shape[0]
    KB = 2
    nch = e // (CHUNK * KB)
    rows_pt = n // NS
    zeros = jnp.zeros((n, d), jnp.float32)

    @functools.partial(
        pl.kernel,
        out_type=jax.ShapeDtypeStruct((NC, n, d), jnp.float32),
        mesh=_sc_mesh(),
        scratch_types=[
            pltpu.VMEM_SHARED((n, d), jnp.float32),
            pltpu.VMEM((KB * CHUNK,), jnp.int32),
            pltpu.VMEM((KB * CHUNK,), jnp.int32),
            pltpu.VMEM((KB * CHUNK, d), jnp.float32),
        ],
    )
    def k(g_hbm, src_hbm, dst_hbm, z_hbm, out_hbm, acc, sidx, didx, rows):
        c = lax.axis_index("c")
        s = lax.axis_index("s")
        w = s * NC + c
        pltpu.sync_copy(
            z_hbm.at[pl.ds(s * rows_pt, rows_pt), :],
            acc.at[pl.ds(s * rows_pt, rows_pt), :],
        )
        plsc.subcore_barrier()
        nj = (nch - w + NW - 1) // NW

        def body(j, carry):
            base = (w + j * NW) * CHUNK * KB
            pltpu.sync_copy(src_hbm.at[pl.ds(base, CHUNK * KB)], sidx)
            pltpu.sync_copy(dst_hbm.at[pl.ds(base, CHUNK * KB)], didx)
            pltpu.sync_copy(g_hbm.at[sidx], rows)
            pltpu.sync_copy(rows, acc.at[didx], add=True)
            return carry

        lax.fori_loop(0, nj, body, 0)

        plsc.subcore_barrier()
        pltpu.sync_copy(
            acc.at[pl.ds(s * rows_pt, rows_pt), :],
            out_hbm.at[c, pl.ds(s * rows_pt, rows_pt), :],
        )

    return k(g, src_i32, dst_i32, zeros)


def _sc_gather2(emb, src_i32, dst_i32):
    n, d = emb.shape
    e = src_i32.shape[0]
    KB = 2
    nch = e // (CHUNK * KB)

    @functools.partial(
        pl.kernel,
        out_type=jax.ShapeDtypeStruct((2, e, d), emb.dtype),
        mesh=_sc_mesh(),
        scratch_types=[
            pltpu.VMEM((KB * CHUNK,), jnp.int32),
            pltpu.VMEM((KB * CHUNK,), jnp.int32),
            pltpu.VMEM((KB * CHUNK, d), emb.dtype),
        ],
    )
    def k(emb_hbm, src_hbm, dst_hbm, out_hbm, sidx, didx, rows):
        c = lax.axis_index("c")
        s = lax.axis_index("s")
        w = s * NC + c
        nj = (nch - w + NW - 1) // NW

        def body(j, carry):
            base = (w + j * NW) * CHUNK * KB
            pltpu.sync_copy(src_hbm.at[pl.ds(base, CHUNK * KB)], sidx)
            pltpu.sync_copy(dst_hbm.at[pl.ds(base, CHUNK * KB)], didx)
            pltpu.sync_copy(emb_hbm.at[sidx], rows)
            pltpu.sync_copy(rows, out_hbm.at[0, pl.ds(base, CHUNK * KB), :])
            pltpu.sync_copy(emb_hbm.at[didx], rows)
            pltpu.sync_copy(rows, out_hbm.at[1, pl.ds(base, CHUNK * KB), :])
            return carry

        lax.fori_loop(0, nj, body, 0)

    return k(emb, src_i32, dst_i32)


def _tc_pre(degp, x, w):
    n, d = x.shape
    h = w.shape[1]

    def body(degp_ref, x_ref, w_ref, dis_ref, g_ref):
        deg = 1.0 + degp_ref[0] + degp_ref[1]
        dis = lax.rsqrt(deg)
        dis_ref[...] = dis
        g_ref[...] = dis * jnp.dot(
            x_ref[...], w_ref[...], preferred_element_type=jnp.float32
        )

    return pl.pallas_call(
        body,
        out_shape=[
            jax.ShapeDtypeStruct((n, 1), jnp.float32),
            jax.ShapeDtypeStruct((n, h), jnp.float32),
        ],
    )(degp, x, w)


def _tc_mid(accp, g, dis, b, w_next):
    n, d = g.shape
    h_dim = w_next.shape[1]

    def body(accp_ref, g_ref, dis_ref, b_ref, w_ref, gn_ref):
        dis = dis_ref[...]
        hval = jnp.maximum(
            dis * (accp_ref[0] + accp_ref[1] + g_ref[...]) + b_ref[...], 0.0
        )
        gn_ref[...] = dis * jnp.dot(
            hval, w_ref[...], preferred_element_type=jnp.float32
        )

    return pl.pallas_call(
        body,
        out_shape=jax.ShapeDtypeStruct((n, h_dim), jnp.float32),
    )(accp, g, dis, b, w_next)


def _tc_final(accp, g, dis, b):
    n, d = g.shape

    def body(accp_ref, g_ref, dis_ref, b_ref, emb_ref):
        dis = dis_ref[...]
        hval = jnp.maximum(
            dis * (accp_ref[0] + accp_ref[1] + g_ref[...]) + b_ref[...], 0.0
        )
        hval = jnp.where(jnp.isnan(hval), 0.0, hval)
        emb_ref[...] = hval

    return pl.pallas_call(
        body,
        out_shape=jax.ShapeDtypeStruct((n, d), jnp.float32),
    )(accp, g, dis, b)


def _tc_scorer(gfeat, m1, mb1, m2, mb2, m3, mb3):
    _, e, dp = gfeat.shape
    be = 1280
    grid = e // be

    def body(g_ref, m1_ref, mb1_ref, m2_ref, mb2_ref, m3_ref, mb3_ref, s_ref):
        ga = g_ref[0].astype(jnp.bfloat16)
        gb = g_ref[1].astype(jnp.bfloat16)
        s1 = jnp.dot(ga, m1_ref[0], preferred_element_type=jnp.float32)
        s1 = s1 + jnp.dot(gb, m1_ref[1], preferred_element_type=jnp.float32)
        s1 = jnp.maximum(s1 + mb1_ref[...], 0.0).astype(jnp.bfloat16)
        s2 = jnp.maximum(
            jnp.dot(s1, m2_ref[...], preferred_element_type=jnp.float32)
            + mb2_ref[...],
            0.0,
        )
        s3 = (
            jnp.dot(s2, m3_ref[...], preferred_element_type=jnp.float32)
            + mb3_ref[...]
        )
        s3 = jnp.clip(s3, -1000000.0, 1000000.0)
        s_ref[...] = jnp.where(jnp.isnan(s3), 0.0, s3)

    return pl.pallas_call(
        body,
        grid=(grid,),
        in_specs=[
            pl.BlockSpec((2, be, dp), lambda i: (0, i, 0)),
            pl.BlockSpec(m1.shape, lambda i: (0, 0, 0)),
            pl.BlockSpec(mb1.shape, lambda i: (0,)),
            pl.BlockSpec(m2.shape, lambda i: (0, 0)),
            pl.BlockSpec(mb2.shape, lambda i: (0,)),
            pl.BlockSpec(m3.shape, lambda i: (0, 0)),
            pl.BlockSpec(mb3.shape, lambda i: (0,)),
        ],
        out_specs=pl.BlockSpec((be, 1), lambda i: (i, 0)),
        out_shape=jax.ShapeDtypeStruct((e, 1), jnp.float32),
    )(gfeat, m1, mb1, m2, mb2, m3, mb3)


def kernel(x, edge_index, W1, b1, W2, b2, W3, b3, M1, mb1, M2, mb2, M3, mb3):
    n = x.shape[0]
    e = edge_index.shape[1]
    np_ = ((n + NS * 16 - 1) // (NS * 16)) * (NS * 16)
    src = edge_index[0].astype(jnp.int32)
    dst = edge_index[1].astype(jnp.int32)
    xp = jnp.pad(x, ((0, np_ - n), (0, 0)))

    degp = _sc_degree(dst, np_).reshape(NC, np_, 1)
    dis, g1 = _tc_pre(degp, xp, W1)

    a1 = _sc_segsum(g1, src, dst)
    g2 = _tc_mid(a1, g1, dis, b1, W2)

    a2 = _sc_segsum(g2, src, dst)
    g3 = _tc_mid(a2, g2, dis, b2, W3)

    a3 = _sc_segsum(g3, src, dst)
    emb = _tc_final(a3, g3, dis, b3)

    h2 = M1.shape[0] // 2
    m1s = jnp.stack([M1[:h2], M1[h2:]]).astype(jnp.bfloat16)
    nsc = 5
    ec = e // nsc
    parts = []
    for k in range(nsc):
        gf = _sc_gather2(emb, src[k * ec : (k + 1) * ec], dst[k * ec : (k + 1) * ec])
        parts.append(_tc_scorer(gf, m1s, mb1, M2.astype(jnp.bfloat16), mb2, M3, mb3))
    s = jnp.concatenate(parts, axis=0)

    return (s[:e, 0], emb[:n])


# speedup vs baseline: 2.9891x; 1.0058x over previous
import functools

import jax
import jax.numpy as jnp
from jax import lax
from jax.experimental import pallas as pl
from jax.experimental.pallas import tpu as pltpu
from jax.experimental.pallas import tpu_sc as plsc

NC = 2
NS = 16
NW = NC * NS
CHUNK = 128


def _sc_mesh():
    return plsc.VectorSubcoreMesh(
        core_axis_name="c", subcore_axis_name="s", num_cores=NC, num_subcores=NS
    )


def _sc_degree(dst_i32, n):
    e = dst_i32.shape[0]
    KB = 2
    nch = e // (CHUNK * KB)
    rows_pt = n // NS
    zeros = jnp.zeros((n,), jnp.float32)
    ones = jnp.ones((KB * CHUNK,), jnp.float32)

    @functools.partial(
        pl.kernel,
        out_type=jax.ShapeDtypeStruct((NC * n,), jnp.float32),
        mesh=_sc_mesh(),
        scratch_types=[
            pltpu.VMEM_SHARED((n,), jnp.float32),
            pltpu.VMEM((KB * CHUNK,), jnp.int32),
            pltpu.VMEM((KB * CHUNK,), jnp.float32),
        ],
    )
    def k(dst_hbm, z_hbm, ones_hbm, out_hbm, acc, didx, vals):
        c = lax.axis_index("c")
        s = lax.axis_index("s")
        w = s * NC + c
        pltpu.sync_copy(ones_hbm, vals)
        pltpu.sync_copy(
            z_hbm.at[pl.ds(s * rows_pt, rows_pt)],
            acc.at[pl.ds(s * rows_pt, rows_pt)],
        )
        plsc.subcore_barrier()
        nj = (nch - w + NW - 1) // NW

        def body(j, carry):
            base = (w + j * NW) * CHUNK * KB
            pltpu.sync_copy(dst_hbm.at[pl.ds(base, CHUNK * KB)], didx)
            pltpu.sync_copy(vals, acc.at[didx], add=True)
            return carry

        lax.fori_loop(0, nj, body, 0)
        plsc.subcore_barrier()
        pltpu.sync_copy(
            acc.at[pl.ds(s * rows_pt, rows_pt)],
            out_hbm.at[pl.ds(c * n + s * rows_pt, rows_pt)],
        )

    return k(dst_i32, zeros, ones)


def _sc_segsum(g, src_i32, dst_i32):
    n, d = g.shape
    e = src_i32.shape[0]
    KB = 2
    nch = e // (CHUNK * KB)
    rows_pt = n // NS
    zeros = jnp.zeros((n, d), jnp.float32)

    @functools.partial(
        pl.kernel,
        out_type=jax.ShapeDtypeStruct((NC, n, d), jnp.float32),
        mesh=_sc_mesh(),
        scratch_types=[
            pltpu.VMEM_SHARED((n, d), jnp.float32),
            pltpu.VMEM((KB * CHUNK,), jnp.int32),
            pltpu.VMEM((KB * CHUNK,), jnp.int32),
            pltpu.VMEM((KB * CHUNK, d), jnp.float32),
        ],
    )
    def k(g_hbm, src_hbm, dst_hbm, z_hbm, out_hbm, acc, sidx, didx, rows):
        c = lax.axis_index("c")
        s = lax.axis_index("s")
        w = s * NC + c
        pltpu.sync_copy(
            z_hbm.at[pl.ds(s * rows_pt, rows_pt), :],
            acc.at[pl.ds(s * rows_pt, rows_pt), :],
        )
        plsc.subcore_barrier()
        nj = (nch - w + NW - 1) // NW

        def body(j, carry):
            base = (w + j * NW) * CHUNK * KB
            pltpu.sync_copy(src_hbm.at[pl.ds(base, CHUNK * KB)], sidx)
            pltpu.sync_copy(dst_hbm.at[pl.ds(base, CHUNK * KB)], didx)
            pltpu.sync_copy(g_hbm.at[sidx], rows)
            pltpu.sync_copy(rows, acc.at[didx], add=True)
            return carry

        lax.fori_loop(0, nj, body, 0)

        plsc.subcore_barrier()
        pltpu.sync_copy(
            acc.at[pl.ds(s * rows_pt, rows_pt), :],
            out_hbm.at[c, pl.ds(s * rows_pt, rows_pt), :],
        )

    return k(g, src_i32, dst_i32, zeros)


def _sc_gather2(emb, src_i32, dst_i32):
    n, d = emb.shape
    e = src_i32.shape[0]
    KB = 2
    nch = e // (CHUNK * KB)

    @functools.partial(
        pl.kernel,
        out_type=jax.ShapeDtypeStruct((2, e, d), emb.dtype),
        mesh=_sc_mesh(),
        scratch_types=[
            pltpu.VMEM((KB * CHUNK,), jnp.int32),
            pltpu.VMEM((KB * CHUNK,), jnp.int32),
            pltpu.VMEM((KB * CHUNK, d), emb.dtype),
        ],
    )
    def k(emb_hbm, src_hbm, dst_hbm, out_hbm, sidx, didx, rows):
        c = lax.axis_index("c")
        s = lax.axis_index("s")
        w = s * NC + c
        nj = (nch - w + NW - 1) // NW

        def body(j, carry):
            base = (w + j * NW) * CHUNK * KB
            pltpu.sync_copy(src_hbm.at[pl.ds(base, CHUNK * KB)], sidx)
            pltpu.sync_copy(dst_hbm.at[pl.ds(base, CHUNK * KB)], didx)
            pltpu.sync_copy(emb_hbm.at[sidx], rows)
            pltpu.sync_copy(rows, out_hbm.at[0, pl.ds(base, CHUNK * KB), :])
            pltpu.sync_copy(emb_hbm.at[didx], rows)
            pltpu.sync_copy(rows, out_hbm.at[1, pl.ds(base, CHUNK * KB), :])
            return carry

        lax.fori_loop(0, nj, body, 0)

    return k(emb, src_i32, dst_i32)


def _tc_pre(degp, x, w):
    n, d = x.shape
    h = w.shape[1]

    def body(degp_ref, x_ref, w_ref, dis_ref, g_ref):
        deg = 1.0 + degp_ref[0] + degp_ref[1]
        dis = lax.rsqrt(deg)
        dis_ref[...] = dis
        g_ref[...] = dis * jnp.dot(
            x_ref[...], w_ref[...], preferred_element_type=jnp.float32
        )

    return pl.pallas_call(
        body,
        out_shape=[
            jax.ShapeDtypeStruct((n, 1), jnp.float32),
            jax.ShapeDtypeStruct((n, h), jnp.float32),
        ],
    )(degp, x, w)


def _tc_mid(accp, g, dis, b, w_next):
    n, d = g.shape
    h_dim = w_next.shape[1]

    def body(accp_ref, g_ref, dis_ref, b_ref, w_ref, gn_ref):
        dis = dis_ref[...]
        hval = jnp.maximum(
            dis * (accp_ref[0] + accp_ref[1] + g_ref[...]) + b_ref[...], 0.0
        )
        gn_ref[...] = dis * jnp.dot(
            hval, w_ref[...], preferred_element_type=jnp.float32
        )

    return pl.pallas_call(
        body,
        out_shape=jax.ShapeDtypeStruct((n, h_dim), jnp.float32),
    )(accp, g, dis, b, w_next)


def _tc_final(accp, g, dis, b):
    n, d = g.shape

    def body(accp_ref, g_ref, dis_ref, b_ref, emb_ref):
        dis = dis_ref[...]
        hval = jnp.maximum(
            dis * (accp_ref[0] + accp_ref[1] + g_ref[...]) + b_ref[...], 0.0
        )
        hval = jnp.where(jnp.isnan(hval), 0.0, hval)
        emb_ref[...] = hval

    return pl.pallas_call(
        body,
        out_shape=jax.ShapeDtypeStruct((n, d), jnp.float32),
    )(accp, g, dis, b)


def _tc_scorer(gfeat, m1, mb1, m2, mb2, m3, mb3):
    _, e, dp = gfeat.shape
    be = 1280
    grid = e // be

    def body(g_ref, m1_ref, mb1_ref, m2_ref, mb2_ref, m3_ref, mb3_ref, s_ref):
        ga = g_ref[0].astype(jnp.bfloat16)
        gb = g_ref[1].astype(jnp.bfloat16)
        s1 = jnp.dot(ga, m1_ref[0], preferred_element_type=jnp.float32)
        s1 = s1 + jnp.dot(gb, m1_ref[1], preferred_element_type=jnp.float32)
        s1 = jnp.maximum(s1 + mb1_ref[...], 0.0).astype(jnp.bfloat16)
        s2 = jnp.maximum(
            jnp.dot(s1, m2_ref[...], preferred_element_type=jnp.float32)
            + mb2_ref[...],
            0.0,
        )
        s3 = (
            jnp.dot(s2, m3_ref[...], preferred_element_type=jnp.float32)
            + mb3_ref[...]
        )
        s3 = jnp.clip(s3, -1000000.0, 1000000.0)
        s_ref[...] = jnp.where(jnp.isnan(s3), 0.0, s3)

    return pl.pallas_call(
        body,
        grid=(grid,),
        in_specs=[
            pl.BlockSpec((2, be, dp), lambda i: (0, i, 0)),
            pl.BlockSpec(m1.shape, lambda i: (0, 0, 0)),
            pl.BlockSpec(mb1.shape, lambda i: (0,)),
            pl.BlockSpec(m2.shape, lambda i: (0, 0)),
            pl.BlockSpec(mb2.shape, lambda i: (0,)),
            pl.BlockSpec(m3.shape, lambda i: (0, 0)),
            pl.BlockSpec(mb3.shape, lambda i: (0,)),
        ],
        out_specs=pl.BlockSpec((be, 1), lambda i: (i, 0)),
        out_shape=jax.ShapeDtypeStruct((e, 1), jnp.float32),
    )(gfeat, m1, mb1, m2, mb2, m3, mb3)


def kernel(x, edge_index, W1, b1, W2, b2, W3, b3, M1, mb1, M2, mb2, M3, mb3):
    n = x.shape[0]
    e = edge_index.shape[1]
    np_ = ((n + NS * 16 - 1) // (NS * 16)) * (NS * 16)
    src = edge_index[0].astype(jnp.int32)
    dst = edge_index[1].astype(jnp.int32)
    xp = jnp.pad(x, ((0, np_ - n), (0, 0)))

    degp = _sc_degree(dst, np_).reshape(NC, np_, 1)
    dis, g1 = _tc_pre(degp, xp, W1)

    a1 = _sc_segsum(g1, src, dst)
    g2 = _tc_mid(a1, g1, dis, b1, W2)

    a2 = _sc_segsum(g2, src, dst)
    g3 = _tc_mid(a2, g2, dis, b2, W3)

    a3 = _sc_segsum(g3, src, dst)
    emb = _tc_final(a3, g3, dis, b3)

    h2 = M1.shape[0] // 2
    m1s = jnp.stack([M1[:h2], M1[h2:]]).astype(jnp.bfloat16)
    nsc = 10
    ec = e // nsc
    parts = []
    for k in range(nsc):
        gf = _sc_gather2(emb, src[k * ec : (k + 1) * ec], dst[k * ec : (k + 1) * ec])
        parts.append(_tc_scorer(gf, m1s, mb1, M2.astype(jnp.bfloat16), mb2, M3, mb3))
    s = jnp.concatenate(parts, axis=0)

    return (s[:e, 0], emb[:n])
